# baseline (speedup 1.0000x reference)
import functools

import jax
import jax.numpy as jnp
from jax import lax
from jax.experimental import pallas as pl
from jax.experimental.pallas import tpu as pltpu
from jax.experimental.pallas import tpu_sc as plsc

N = 10000
NP = 10240
E = 160000
D = 256
DH = 128
B = 128
NCHUNK = E // B
NTILES = 16
ROWS_PER_TILE = NP // NTILES
ZROWS = B

NCH_BASE = NCHUNK // NTILES
NCH_REM = NCHUNK % NTILES
NCH_CEIL = NCH_BASE + (2 if NCH_REM else 0)

_mesh = plsc.VectorSubcoreMesh(core_axis_name="c", subcore_axis_name="s")


H = B // 2


def _sc_agg_body(x3, srcx_h, dsth_h, w_h, agg3, ws_out,
                 srci_a, srci_b, dsti_a, dsti_b, w_a, w_b,
                 rows_a, rows_b, zws_v,
                 acc_sh, ws_sh, sem_a, sem_b, ssem_a, ssem_b):
    c = lax.axis_index("c")
    s = lax.axis_index("s")

    def zrow(i, carry):
        for k in range(DH // 16):
            rows_a[i, k * 16:(k + 1) * 16] = jnp.zeros((16,), jnp.float32)
        return carry
    lax.fori_loop(0, ZROWS, zrow, 0)

    def zws_row(i, carry):
        zws_v[pl.ds(i * 16, 16)] = jnp.zeros((16,), jnp.float32)
        return carry
    lax.fori_loop(0, ROWS_PER_TILE // 16, zws_row, 0)

    base_rows = s * ROWS_PER_TILE
    zdescs = [pltpu.async_copy(
        rows_a, acc_sh.at[pl.ds(base_rows + kk * ZROWS, ZROWS)], sem_a)
        for kk in range(ROWS_PER_TILE // ZROWS)]
    for zd in zdescs:
        zd.wait()
    pltpu.sync_copy(zws_v, ws_sh.at[pl.ds(base_rows, ROWS_PER_TILE)])

    plsc.subcore_barrier()

    nch = NCH_BASE + jnp.where(s < NCH_REM, 1, 0)
    bufs = ((srci_a, dsti_a, w_a, rows_a, sem_a, ssem_a),
            (srci_b, dsti_b, w_b, rows_b, sem_b, ssem_b))

    def issue(jj, buf):
        srci_v, dsti_v, w_v, rows_v, sem, _ = buf
        cid = s + jj * NTILES
        base = cid * B
        pltpu.sync_copy(srcx_h.at[pl.ds(base, B)], srci_v)
        pltpu.sync_copy(dsth_h.at[cid], dsti_v)
        pltpu.sync_copy(w_h.at[pl.ds(base, B)], w_v)
        pltpu.async_copy(x3.at[c].at[srci_v], rows_v, sem)

    issue(0, bufs[0])
    issue(1, bufs[1])

    @pl.loop(0, NCH_CEIL, step=2)
    def _chunks(j):
        for bsel in range(2):
            srci_v, dsti_v, w_v, rows_v, sem, ssem = bufs[bsel]
            jj = j + bsel

            @pl.when(jj < nch)
            def _():
                pltpu.make_async_copy(
                    x3.at[c].at[srci_v], rows_v, sem).wait()

                def grp(g, rcarry):
                    wvec = w_v[pl.ds(g * 16, 16)]
                    for r in range(16):
                        wr = wvec[r]
                        row = g * 16 + r
                        for k in range(DH // 16):
                            sl = pl.ds(k * 16, 16)
                            rows_v[row, sl] = rows_v[row, sl] * wr
                    return rcarry

                lax.fori_loop(0, H // 16, grp, 0)
                pltpu.async_copy(rows_v.at[pl.ds(0, H)],
                                 acc_sh.at[dsti_v.at[0]], ssem, add=True)
                lax.fori_loop(H // 16, B // 16, grp, 0)
                pltpu.sync_copy(rows_v.at[pl.ds(H, H)],
                                acc_sh.at[dsti_v.at[1]], add=True)
                pltpu.make_async_copy(rows_v.at[pl.ds(0, H)],
                                      acc_sh.at[dsti_v.at[0]], ssem).wait()

                @pl.when((jj & 1) == c)
                def _():
                    pltpu.sync_copy(w_v.at[pl.ds(0, H)],
                                    ws_sh.at[dsti_v.at[0]], add=True)
                    pltpu.sync_copy(w_v.at[pl.ds(H, H)],
                                    ws_sh.at[dsti_v.at[1]], add=True)

                @pl.when(jj + 2 < nch)
                def _():
                    issue(jj + 2, bufs[bsel])

    plsc.subcore_barrier()

    pltpu.sync_copy(acc_sh.at[pl.ds(base_rows, ROWS_PER_TILE)],
                    agg3.at[c].at[pl.ds(base_rows, ROWS_PER_TILE)])
    pltpu.sync_copy(ws_sh.at[pl.ds(base_rows, ROWS_PER_TILE)],
                    ws_out.at[c].at[pl.ds(base_rows, ROWS_PER_TILE)])


_sc_agg = functools.partial(
    pl.kernel,
    out_type=(jax.ShapeDtypeStruct((2, NP, DH), jnp.float32),
              jax.ShapeDtypeStruct((2, NP), jnp.float32)),
    mesh=_mesh,
    scratch_types=[
        pltpu.VMEM((B,), jnp.int32),
        pltpu.VMEM((B,), jnp.int32),
        pltpu.VMEM((2, B // 2), jnp.int32),
        pltpu.VMEM((2, B // 2), jnp.int32),
        pltpu.VMEM((B,), jnp.float32),
        pltpu.VMEM((B,), jnp.float32),
        pltpu.VMEM((B, DH), jnp.float32),
        pltpu.VMEM((B, DH), jnp.float32),
        pltpu.VMEM((ROWS_PER_TILE,), jnp.float32),
        pltpu.VMEM_SHARED((NP, DH), jnp.float32),
        pltpu.VMEM_SHARED((NP,), jnp.float32),
        pltpu.SemaphoreType.DMA,
        pltpu.SemaphoreType.DMA,
        pltpu.SemaphoreType.DMA,
        pltpu.SemaphoreType.DMA,
    ],
)(_sc_agg_body)


def _dense_body(h_ref, agg_ref, ws_ref, W_ref, b_ref, out_ref):
    hl = h_ref[0]
    hh = h_ref[1]
    inv = 1.0 / (ws_ref[0] + ws_ref[1] + 1e-9)
    al = agg_ref[0] * inv
    ah = agg_ref[1] * inv
    W = W_ref[...]
    z = (jnp.dot(hl, W[0:128, :], preferred_element_type=jnp.float32)
         + jnp.dot(hh, W[128:256, :], preferred_element_type=jnp.float32)
         + jnp.dot(al, W[256:384, :], preferred_element_type=jnp.float32)
         + jnp.dot(ah, W[384:512, :], preferred_element_type=jnp.float32)
         + b_ref[...])
    z = jnp.maximum(z, 0.0)
    z = z / (jnp.sqrt(jnp.sum(z * z, axis=1, keepdims=True)) + 1e-9)
    out_ref[0, :, :] = z[:, :DH]
    out_ref[1, :, :] = z[:, DH:]


_R = 256

_dense = pl.pallas_call(
    _dense_body,
    grid=(NP // _R,),
    in_specs=[
        pl.BlockSpec((2, _R, DH), lambda i: (0, i, 0)),
        pl.BlockSpec((2, _R, DH), lambda i: (0, i, 0)),
        pl.BlockSpec((2, _R, 1), lambda i: (0, i, 0)),
        pl.BlockSpec((2 * D, D), lambda i: (0, 0)),
        pl.BlockSpec((1, D), lambda i: (0, 0)),
    ],
    out_specs=pl.BlockSpec((2, _R, DH), lambda i: (0, i, 0)),
    out_shape=jax.ShapeDtypeStruct((2, NP, DH), jnp.float32),
)


def kernel(x, edge_index, edge_weight, W0, b0, W1, b1):
    h3 = jnp.pad(x, ((0, NP - N), (0, 0))).reshape(NP, 2, DH).transpose(1, 0, 2)
    dst3 = edge_index[1].reshape(NCHUNK, 2, B // 2)
    for W, b in ((W0, b0), (W1, b1)):
        agg3, ws = _sc_agg(h3, edge_index[0], dst3, edge_weight)
        h3 = _dense(h3, agg3, ws.reshape(2, NP, 1), W, b.reshape(1, D))
    return h3.transpose(1, 0, 2).reshape(NP, D)[:N]


# speedup vs baseline: 1.2799x; 1.2799x over previous
import functools

import jax
import jax.numpy as jnp
from jax import lax
from jax.experimental import pallas as pl
from jax.experimental.pallas import tpu as pltpu
from jax.experimental.pallas import tpu_sc as plsc

N = 10000
NP = 10240
E = 160000
D = 256
DH = 128
B = 128
NCHUNK = E // B
NTILES = 16
ROWS_PER_TILE = NP // NTILES
ZROWS = B

NCH_BASE = NCHUNK // NTILES
NCH_REM = NCHUNK % NTILES
NCH_CEIL = NCH_BASE + (2 if NCH_REM else 0)

_mesh = plsc.VectorSubcoreMesh(core_axis_name="c", subcore_axis_name="s")


def _sc_agg_body(x3, ei_h, agg3, ws_out,
                 idx3_a, idx3_b, w_a, w_b, rows_a, rows_b, zws_v,
                 acc_sh, ws_sh, sem_a, sem_b):
    c = lax.axis_index("c")
    s = lax.axis_index("s")

    def zrow(i, carry):
        for k in range(DH // 16):
            rows_a[i, k * 16:(k + 1) * 16] = jnp.zeros((16,), jnp.float32)
        return carry
    lax.fori_loop(0, ZROWS, zrow, 0)

    def zws_row(i, carry):
        zws_v[pl.ds(i * 16, 16)] = jnp.zeros((16,), jnp.float32)
        return carry
    lax.fori_loop(0, ROWS_PER_TILE // 16, zws_row, 0)

    base_rows = s * ROWS_PER_TILE
    for kk in range(ROWS_PER_TILE // ZROWS):
        pltpu.sync_copy(rows_a, acc_sh.at[pl.ds(base_rows + kk * ZROWS, ZROWS)])
    pltpu.sync_copy(zws_v, ws_sh.at[pl.ds(base_rows, ROWS_PER_TILE)])

    plsc.subcore_barrier()

    nch = NCH_BASE + jnp.where(s < NCH_REM, 1, 0)
    bufs = ((idx3_a, w_a, rows_a, sem_a), (idx3_b, w_b, rows_b, sem_b))

    def issue(jj, buf):
        idx3_v, w_v, rows_v, sem = buf
        base = (s + jj * NTILES) * B
        pltpu.sync_copy(ei_h.at[:, pl.ds(base, B)], idx3_v)
        pltpu.async_copy(x3.at[c].at[idx3_v.at[0]], rows_v, sem)

    issue(0, bufs[0])
    issue(1, bufs[1])

    @pl.loop(0, NCH_CEIL, step=2)
    def _chunks(j):
        for bsel in range(2):
            idx3_v, w_v, rows_v, sem = bufs[bsel]
            jj = j + bsel

            @pl.when(jj < nch)
            def _():
                pltpu.make_async_copy(
                    x3.at[c].at[idx3_v.at[0]], rows_v, sem).wait()

                def grp(g, rcarry):
                    wvec = lax.bitcast_convert_type(
                        idx3_v[2, pl.ds(g * 16, 16)], jnp.float32)
                    w_v[pl.ds(g * 16, 16)] = wvec
                    for r in range(16):
                        wr = wvec[r]
                        row = g * 16 + r
                        for k in range(DH // 16):
                            sl = pl.ds(k * 16, 16)
                            rows_v[row, sl] = rows_v[row, sl] * wr
                    return rcarry
                lax.fori_loop(0, B // 16, grp, 0)

                pltpu.sync_copy(rows_v, acc_sh.at[idx3_v.at[1]], add=True)

                @pl.when((jj & 1) == c)
                def _():
                    pltpu.sync_copy(w_v, ws_sh.at[idx3_v.at[1]], add=True)

                @pl.when(jj + 2 < nch)
                def _():
                    issue(jj + 2, bufs[bsel])

    plsc.subcore_barrier()

    pltpu.sync_copy(acc_sh.at[pl.ds(base_rows, ROWS_PER_TILE)],
                    agg3.at[c].at[pl.ds(base_rows, ROWS_PER_TILE)])
    pltpu.sync_copy(ws_sh.at[pl.ds(base_rows, ROWS_PER_TILE)],
                    ws_out.at[c].at[pl.ds(base_rows, ROWS_PER_TILE)])


_sc_agg = functools.partial(
    pl.kernel,
    out_type=(jax.ShapeDtypeStruct((2, NP, DH), jnp.float32),
              jax.ShapeDtypeStruct((2, NP), jnp.float32)),
    mesh=_mesh,
    scratch_types=[
        pltpu.VMEM((3, B), jnp.int32),
        pltpu.VMEM((3, B), jnp.int32),
        pltpu.VMEM((B,), jnp.float32),
        pltpu.VMEM((B,), jnp.float32),
        pltpu.VMEM((B, DH), jnp.float32),
        pltpu.VMEM((B, DH), jnp.float32),
        pltpu.VMEM((ROWS_PER_TILE,), jnp.float32),
        pltpu.VMEM_SHARED((NP, DH), jnp.float32),
        pltpu.VMEM_SHARED((NP,), jnp.float32),
        pltpu.SemaphoreType.DMA,
        pltpu.SemaphoreType.DMA,
    ],
)(_sc_agg_body)


def _dense_body(h_ref, agg_ref, ws_ref, W_ref, b_ref, out_ref):
    hl = h_ref[0]
    hh = h_ref[1]
    inv = 1.0 / (ws_ref[0] + ws_ref[1] + 1e-9)
    al = agg_ref[0] * inv
    ah = agg_ref[1] * inv
    W = W_ref[...]
    z = (jnp.dot(hl, W[0:128, :], preferred_element_type=jnp.float32)
         + jnp.dot(hh, W[128:256, :], preferred_element_type=jnp.float32)
         + jnp.dot(al, W[256:384, :], preferred_element_type=jnp.float32)
         + jnp.dot(ah, W[384:512, :], preferred_element_type=jnp.float32)
         + b_ref[...])
    z = jnp.maximum(z, 0.0)
    z = z / (jnp.sqrt(jnp.sum(z * z, axis=1, keepdims=True)) + 1e-9)
    out_ref[0, :, :] = z[:, :DH]
    out_ref[1, :, :] = z[:, DH:]


def _dense_last_body(h_ref, agg_ref, ws_ref, W_ref, b_ref, out_ref):
    hl = h_ref[0]
    hh = h_ref[1]
    inv = 1.0 / (ws_ref[0] + ws_ref[1] + 1e-9)
    al = agg_ref[0] * inv
    ah = agg_ref[1] * inv
    W = W_ref[...]
    z = (jnp.dot(hl, W[0:128, :], preferred_element_type=jnp.float32)
         + jnp.dot(hh, W[128:256, :], preferred_element_type=jnp.float32)
         + jnp.dot(al, W[256:384, :], preferred_element_type=jnp.float32)
         + jnp.dot(ah, W[384:512, :], preferred_element_type=jnp.float32)
         + b_ref[...])
    z = jnp.maximum(z, 0.0)
    z = z / (jnp.sqrt(jnp.sum(z * z, axis=1, keepdims=True)) + 1e-9)
    out_ref[...] = z


_R = 256

_dense = pl.pallas_call(
    _dense_body,
    grid=(NP // _R,),
    in_specs=[
        pl.BlockSpec((2, _R, DH), lambda i: (0, i, 0)),
        pl.BlockSpec((2, _R, DH), lambda i: (0, i, 0)),
        pl.BlockSpec((2, _R, 1), lambda i: (0, i, 0)),
        pl.BlockSpec((2 * D, D), lambda i: (0, 0)),
        pl.BlockSpec((1, D), lambda i: (0, 0)),
    ],
    out_specs=pl.BlockSpec((2, _R, DH), lambda i: (0, i, 0)),
    out_shape=jax.ShapeDtypeStruct((2, NP, DH), jnp.float32),
)


_dense_last = pl.pallas_call(
    _dense_last_body,
    grid=(NP // _R,),
    in_specs=[
        pl.BlockSpec((2, _R, DH), lambda i: (0, i, 0)),
        pl.BlockSpec((2, _R, DH), lambda i: (0, i, 0)),
        pl.BlockSpec((2, _R, 1), lambda i: (0, i, 0)),
        pl.BlockSpec((2 * D, D), lambda i: (0, 0)),
        pl.BlockSpec((1, D), lambda i: (0, 0)),
    ],
    out_specs=pl.BlockSpec((_R, D), lambda i: (i, 0)),
    out_shape=jax.ShapeDtypeStruct((NP, D), jnp.float32),
)


def kernel(x, edge_index, edge_weight, W0, b0, W1, b1):
    h3 = jnp.pad(x, ((0, NP - N), (0, 0))).reshape(NP, 2, DH).transpose(1, 0, 2)
    ei3 = jnp.concatenate(
        [edge_index,
         lax.bitcast_convert_type(edge_weight, jnp.int32)[None]], axis=0)

    agg3, ws = _sc_agg(h3, ei3)
    h3 = _dense(h3, agg3, ws.reshape(2, NP, 1), W0, b0.reshape(1, D))
    agg3, ws = _sc_agg(h3, ei3)
    out = _dense_last(h3, agg3, ws.reshape(2, NP, 1), W1, b1.reshape(1, D))
    return out[:N]


# speedup vs baseline: 1.4459x; 1.1297x over previous
import functools

import jax
import jax.numpy as jnp
from jax import lax
from jax.experimental import pallas as pl
from jax.experimental.pallas import tpu as pltpu
from jax.experimental.pallas import tpu_sc as plsc

N = 10000
NP = 10240
E = 160000
D = 256
DH = 128
B = 128
NCHUNK = E // B
NTILES = 16
ROWS_PER_TILE = NP // NTILES
ZROWS = B

NCH_BASE = NCHUNK // NTILES
NCH_REM = NCHUNK % NTILES
NCH_CEIL = NCH_BASE + (2 if NCH_REM else 0)
NCH_CEIL6 = ((NCH_BASE + 1) + 5) // 6 * 6

_mesh = plsc.VectorSubcoreMesh(core_axis_name="c", subcore_axis_name="s")


def _sc_agg_body(x3, ei_h, agg3, ws_out,
                 idx3_0, idx3_1, idx3_2, w_a, w_b, rows_a, rows_b, zws_v,
                 acc_sh, ws_sh, sem_a, sem_b, isem_0, isem_1, isem_2):
    c = lax.axis_index("c")
    s = lax.axis_index("s")

    def zrow(i, carry):
        for k in range(DH // 16):
            rows_a[i, k * 16:(k + 1) * 16] = jnp.zeros((16,), jnp.float32)
        return carry
    lax.fori_loop(0, ZROWS, zrow, 0)

    def zws_row(i, carry):
        zws_v[pl.ds(i * 16, 16)] = jnp.zeros((16,), jnp.float32)
        return carry
    lax.fori_loop(0, ROWS_PER_TILE // 16, zws_row, 0)

    base_rows = s * ROWS_PER_TILE
    for kk in range(ROWS_PER_TILE // ZROWS):
        pltpu.sync_copy(rows_a, acc_sh.at[pl.ds(base_rows + kk * ZROWS, ZROWS)])
    pltpu.sync_copy(zws_v, ws_sh.at[pl.ds(base_rows, ROWS_PER_TILE)])

    plsc.subcore_barrier()

    nch = NCH_BASE + jnp.where(s < NCH_REM, 1, 0)
    rbufs = ((w_a, rows_a, sem_a), (w_b, rows_b, sem_b))
    ibufs = ((idx3_0, isem_0), (idx3_1, isem_1), (idx3_2, isem_2))

    def ei_slice(jj):
        return ei_h.at[:, pl.ds((s + jj * NTILES) * B, B)]

    def issue_idx(jj, ibuf):
        idx3_v, isem = ibuf
        pltpu.async_copy(ei_slice(jj), idx3_v, isem)

    def fire_gather(jj, ibuf, rbuf):
        idx3_v, isem = ibuf
        _, rows_v, sem = rbuf
        pltpu.make_async_copy(ei_slice(jj), idx3_v, isem).wait()
        pltpu.async_copy(x3.at[c].at[idx3_v.at[0]], rows_v, sem)

    issue_idx(0, ibufs[0])
    issue_idx(1, ibufs[1])
    fire_gather(0, ibufs[0], rbufs[0])
    fire_gather(1, ibufs[1], rbufs[1])

    @pl.loop(0, NCH_CEIL6, step=6)
    def _chunks(j):
        for u in range(6):
            w_v, rows_v, sem = rbufs[u % 2]
            idx3_v, isem = ibufs[u % 3]
            i2buf = ibufs[(u + 2) % 3]
            jj = j + u

            @pl.when(jj < nch)
            def _():
                @pl.when(jj + 2 < nch)
                def _():
                    issue_idx(jj + 2, i2buf)

                pltpu.make_async_copy(
                    x3.at[c].at[idx3_v.at[0]], rows_v, sem).wait()

                def grp(g, rcarry):
                    wvec = lax.bitcast_convert_type(
                        idx3_v[2, pl.ds(g * 16, 16)], jnp.float32)
                    w_v[pl.ds(g * 16, 16)] = wvec
                    for r in range(16):
                        wr = wvec[r]
                        row = g * 16 + r
                        for k in range(DH // 16):
                            sl = pl.ds(k * 16, 16)
                            rows_v[row, sl] = rows_v[row, sl] * wr
                    return rcarry
                lax.fori_loop(0, B // 16, grp, 0)

                pltpu.sync_copy(rows_v, acc_sh.at[idx3_v.at[1]], add=True)

                @pl.when((jj & 1) == c)
                def _():
                    pltpu.sync_copy(w_v, ws_sh.at[idx3_v.at[1]], add=True)

                @pl.when(jj + 2 < nch)
                def _():
                    fire_gather(jj + 2, i2buf, rbufs[u % 2])

    plsc.subcore_barrier()

    pltpu.sync_copy(acc_sh.at[pl.ds(base_rows, ROWS_PER_TILE)],
                    agg3.at[c].at[pl.ds(base_rows, ROWS_PER_TILE)])
    pltpu.sync_copy(ws_sh.at[pl.ds(base_rows, ROWS_PER_TILE)],
                    ws_out.at[c].at[pl.ds(base_rows, ROWS_PER_TILE)])


_sc_agg = functools.partial(
    pl.kernel,
    out_type=(jax.ShapeDtypeStruct((2, NP, DH), jnp.float32),
              jax.ShapeDtypeStruct((2, NP), jnp.float32)),
    mesh=_mesh,
    scratch_types=[
        pltpu.VMEM((3, B), jnp.int32),
        pltpu.VMEM((3, B), jnp.int32),
        pltpu.VMEM((3, B), jnp.int32),
        pltpu.VMEM((B,), jnp.float32),
        pltpu.VMEM((B,), jnp.float32),
        pltpu.VMEM((B, DH), jnp.float32),
        pltpu.VMEM((B, DH), jnp.float32),
        pltpu.VMEM((ROWS_PER_TILE,), jnp.float32),
        pltpu.VMEM_SHARED((NP, DH), jnp.float32),
        pltpu.VMEM_SHARED((NP,), jnp.float32),
        pltpu.SemaphoreType.DMA,
        pltpu.SemaphoreType.DMA,
        pltpu.SemaphoreType.DMA,
        pltpu.SemaphoreType.DMA,
        pltpu.SemaphoreType.DMA,
    ],
)(_sc_agg_body)


def _dense_body(h_ref, agg_ref, ws_ref, W_ref, b_ref, out_ref):
    hl = h_ref[0]
    hh = h_ref[1]
    inv = 1.0 / (ws_ref[0] + ws_ref[1] + 1e-9)
    al = agg_ref[0] * inv
    ah = agg_ref[1] * inv
    W = W_ref[...]
    z = (jnp.dot(hl, W[0:128, :], preferred_element_type=jnp.float32)
         + jnp.dot(hh, W[128:256, :], preferred_element_type=jnp.float32)
         + jnp.dot(al, W[256:384, :], preferred_element_type=jnp.float32)
         + jnp.dot(ah, W[384:512, :], preferred_element_type=jnp.float32)
         + b_ref[...])
    z = jnp.maximum(z, 0.0)
    z = z / (jnp.sqrt(jnp.sum(z * z, axis=1, keepdims=True)) + 1e-9)
    out_ref[0, :, :] = z[:, :DH]
    out_ref[1, :, :] = z[:, DH:]


def _dense_last_body(h_ref, agg_ref, ws_ref, W_ref, b_ref, out_ref):
    hl = h_ref[0]
    hh = h_ref[1]
    inv = 1.0 / (ws_ref[0] + ws_ref[1] + 1e-9)
    al = agg_ref[0] * inv
    ah = agg_ref[1] * inv
    W = W_ref[...]
    z = (jnp.dot(hl, W[0:128, :], preferred_element_type=jnp.float32)
         + jnp.dot(hh, W[128:256, :], preferred_element_type=jnp.float32)
         + jnp.dot(al, W[256:384, :], preferred_element_type=jnp.float32)
         + jnp.dot(ah, W[384:512, :], preferred_element_type=jnp.float32)
         + b_ref[...])
    z = jnp.maximum(z, 0.0)
    z = z / (jnp.sqrt(jnp.sum(z * z, axis=1, keepdims=True)) + 1e-9)
    out_ref[...] = z


_R = 256

_dense = pl.pallas_call(
    _dense_body,
    grid=(NP // _R,),
    in_specs=[
        pl.BlockSpec((2, _R, DH), lambda i: (0, i, 0)),
        pl.BlockSpec((2, _R, DH), lambda i: (0, i, 0)),
        pl.BlockSpec((2, _R, 1), lambda i: (0, i, 0)),
        pl.BlockSpec((2 * D, D), lambda i: (0, 0)),
        pl.BlockSpec((1, D), lambda i: (0, 0)),
    ],
    out_specs=pl.BlockSpec((2, _R, DH), lambda i: (0, i, 0)),
    out_shape=jax.ShapeDtypeStruct((2, NP, DH), jnp.float32),
)


_dense_last = pl.pallas_call(
    _dense_last_body,
    grid=(NP // _R,),
    in_specs=[
        pl.BlockSpec((2, _R, DH), lambda i: (0, i, 0)),
        pl.BlockSpec((2, _R, DH), lambda i: (0, i, 0)),
        pl.BlockSpec((2, _R, 1), lambda i: (0, i, 0)),
        pl.BlockSpec((2 * D, D), lambda i: (0, 0)),
        pl.BlockSpec((1, D), lambda i: (0, 0)),
    ],
    out_specs=pl.BlockSpec((_R, D), lambda i: (i, 0)),
    out_shape=jax.ShapeDtypeStruct((NP, D), jnp.float32),
)


def kernel(x, edge_index, edge_weight, W0, b0, W1, b1):
    h3 = jnp.pad(x, ((0, NP - N), (0, 0))).reshape(NP, 2, DH).transpose(1, 0, 2)
    ei3 = jnp.concatenate(
        [edge_index,
         lax.bitcast_convert_type(edge_weight, jnp.int32)[None]], axis=0)

    agg3, ws = _sc_agg(h3, ei3)
    h3 = _dense(h3, agg3, ws.reshape(2, NP, 1), W0, b0.reshape(1, D))
    agg3, ws = _sc_agg(h3, ei3)
    out = _dense_last(h3, agg3, ws.reshape(2, NP, 1), W1, b1.reshape(1, D))
    return out[:N]
